# Initial kernel scaffold; baseline (speedup 1.0000x reference)
import functools

import jax
import jax.numpy as jnp
from jax import lax
from jax.experimental import pallas as pl
from jax.experimental.pallas import tpu as pltpu
from jax.experimental.pallas import tpu_sc as plsc

N_SEG = 512
TILE = 2048
NSC = 16
LANES = 16
ACC = N_SEG + LANES


def _tc_body(coord_ref, elems_ref, w1_ref, b1_ref, w2_ref, b2_ref, e_ref):
    x = coord_ref[...]
    h = jnp.tanh(
        jnp.dot(x, w1_ref[...], preferred_element_type=jnp.float32)
        + b1_ref[...]
    )
    e2 = jnp.dot(h, w2_ref[...], preferred_element_type=jnp.float32)
    e = e2[:, 0] + b2_ref[...]
    el = elems_ref[...]
    dress = (
        jnp.where(el == 1, jnp.float32(-0.5), jnp.float32(0.0))
        + jnp.where(el == 8, jnp.float32(-75.0), jnp.float32(0.0))
    )
    e_ref[...] = e + dress


def _per_atom_energy(coord_p, elems_p, W1, b1, W2, b2):
    n_pad = coord_p.shape[0]
    hidden = W1.shape[1]
    grid = n_pad // TILE
    return pl.pallas_call(
        _tc_body,
        grid=(grid,),
        in_specs=[
            pl.BlockSpec((TILE, 3), lambda i: (i, 0)),
            pl.BlockSpec((TILE,), lambda i: (i,)),
            pl.BlockSpec((3, hidden), lambda i: (0, 0)),
            pl.BlockSpec((hidden,), lambda i: (0,)),
            pl.BlockSpec((hidden, 1), lambda i: (0, 0)),
            pl.BlockSpec((1,), lambda i: (0,)),
        ],
        out_specs=pl.BlockSpec((TILE,), lambda i: (i,)),
        out_shape=jax.ShapeDtypeStruct((n_pad,), jnp.float32),
    )(coord_p, elems_p, W1, b1, W2, b2)


def _sc_body(e_hbm, ids_hbm, out_hbm, e_v, ids_v, acc_v, shared, big_v, tot_v):
    chunk = e_v.shape[0]
    wid = lax.axis_index("s")
    base = wid * chunk
    pltpu.sync_copy(e_hbm.at[pl.ds(base, chunk)], e_v)
    pltpu.sync_copy(ids_hbm.at[pl.ds(base, chunk)], ids_v)

    zero = jnp.zeros((LANES,), jnp.float32)
    for j in range(ACC // LANES):
        acc_v[pl.ds(j * LANES, LANES)] = zero

    def scatter_step(j, carry):
        off = j * LANES
        idx = ids_v[pl.ds(off, LANES)]
        val = e_v[pl.ds(off, LANES)]
        plsc.addupdate_scatter(acc_v, [idx], val)
        return carry

    lax.fori_loop(0, chunk // LANES, scatter_step, 0)

    pltpu.sync_copy(acc_v, shared.at[wid])
    plsc.subcore_barrier()

    @pl.when(wid == 0)
    def _():
        pltpu.sync_copy(shared, big_v)
        for ch in range(N_SEG // LANES):
            s16 = big_v[0, pl.ds(ch * LANES, LANES)]
            for r in range(1, NSC):
                s16 = s16 + big_v[r, pl.ds(ch * LANES, LANES)]
            tot_v[pl.ds(ch * LANES, LANES)] = s16
        pltpu.sync_copy(tot_v, out_hbm)


def _segment_sum_sc(e_p, ids_p):
    n_pad = e_p.shape[0]
    chunk = n_pad // NSC
    mesh = plsc.VectorSubcoreMesh(
        core_axis_name="c", subcore_axis_name="s", num_cores=1
    )
    run = functools.partial(
        pl.kernel,
        out_type=jax.ShapeDtypeStruct((N_SEG,), jnp.float32),
        mesh=mesh,
        scratch_types=[
            pltpu.VMEM((chunk,), jnp.float32),
            pltpu.VMEM((chunk,), jnp.int32),
            pltpu.VMEM((ACC,), jnp.float32),
            pltpu.VMEM_SHARED((NSC, ACC), jnp.float32),
            pltpu.VMEM((NSC, ACC), jnp.float32),
            pltpu.VMEM((N_SEG,), jnp.float32),
        ],
    )(_sc_body)
    return run(e_p, ids_p)


def kernel(ind_1, elems, coord, W1, b1, W2, b2):
    n = coord.shape[0]
    ids = ind_1.reshape(-1).astype(jnp.int32)
    n_pad = -(-n // TILE) * TILE
    pad = n_pad - n
    coord_p = jnp.pad(coord, ((0, pad), (0, 0)))
    elems_p = jnp.pad(elems.astype(jnp.int32), (0, pad))
    ids_p = jnp.pad(ids, (0, pad), constant_values=N_SEG)
    e_p = _per_atom_energy(coord_p, elems_p, W1, b1, W2, b2)
    return _segment_sum_sc(e_p, ids_p)


# speedup vs baseline: 1.9675x; 1.9675x over previous
import functools

import jax
import jax.numpy as jnp
from jax import lax
from jax.experimental import pallas as pl
from jax.experimental.pallas import tpu as pltpu
from jax.experimental.pallas import tpu_sc as plsc

N_SEG = 512
TILE = 2048
NSC = 16
LANES = 16
ACC = N_SEG + LANES


def _tc_body(coord_ref, elems_ref, w1_ref, b1_ref, w2_ref, b2_ref, e_ref):
    x = coord_ref[...]
    h = jnp.tanh(
        jnp.dot(x, w1_ref[...], preferred_element_type=jnp.float32)
        + b1_ref[...]
    )
    e2 = jnp.dot(h, w2_ref[...], preferred_element_type=jnp.float32)
    e = e2[:, 0] + b2_ref[...]
    el = elems_ref[...]
    dress = (
        jnp.where(el == 1, jnp.float32(-0.5), jnp.float32(0.0))
        + jnp.where(el == 8, jnp.float32(-75.0), jnp.float32(0.0))
    )
    e_ref[...] = e + dress


def _per_atom_energy(coord_p, elems_p, W1, b1, W2, b2):
    n_pad = coord_p.shape[0]
    hidden = W1.shape[1]
    grid = n_pad // TILE
    return pl.pallas_call(
        _tc_body,
        grid=(grid,),
        in_specs=[
            pl.BlockSpec((TILE, 3), lambda i: (i, 0)),
            pl.BlockSpec((TILE,), lambda i: (i,)),
            pl.BlockSpec((3, hidden), lambda i: (0, 0)),
            pl.BlockSpec((hidden,), lambda i: (0,)),
            pl.BlockSpec((hidden, 1), lambda i: (0, 0)),
            pl.BlockSpec((1,), lambda i: (0,)),
        ],
        out_specs=pl.BlockSpec((TILE,), lambda i: (i,)),
        out_shape=jax.ShapeDtypeStruct((n_pad,), jnp.float32),
    )(coord_p, elems_p, W1, b1, W2, b2)


def _sc_body(e_hbm, ids_hbm, out_hbm, e_v, ids_v, acc_v, shared, big_v, tot_v):
    chunk = e_v.shape[0]
    wid = lax.axis_index("s")
    base = wid * chunk
    pltpu.sync_copy(e_hbm.at[pl.ds(base, chunk)], e_v)
    pltpu.sync_copy(ids_hbm.at[pl.ds(base, chunk)], ids_v.at[pl.ds(0, chunk)])

    zero = jnp.zeros((LANES,), jnp.float32)
    for j in range(ACC // LANES):
        acc_v[pl.ds(j * LANES, LANES)] = zero

    iota = lax.iota(jnp.int32, LANES)
    last = LANES - 1

    def scatter_step(j, carry):
        off = j * LANES
        idc = ids_v[pl.ds(off, LANES)]
        idn = ids_v[pl.ds(off + 1, LANES)]
        val = e_v[pl.ds(off, LANES)]
        p = plsc.cumsum(val)
        bnd = (idc != idn) & (iota < last)
        end_mask = bnd | (iota == last)
        plsc.addupdate_scatter(acc_v, [idc], p, mask=end_mask)
        plsc.addupdate_scatter(acc_v, [idn], -p, mask=bnd)
        return carry

    lax.fori_loop(0, chunk // LANES, scatter_step, 0)

    pltpu.sync_copy(acc_v, shared.at[pl.ds(wid * ACC, ACC)])
    plsc.subcore_barrier()

    @pl.when(wid == 0)
    def _():
        pltpu.sync_copy(shared, big_v)
        for ch in range(N_SEG // LANES):
            s16 = big_v[pl.ds(ch * LANES, LANES)]
            for r in range(1, NSC):
                s16 = s16 + big_v[pl.ds(r * ACC + ch * LANES, LANES)]
            tot_v[pl.ds(ch * LANES, LANES)] = s16
        pltpu.sync_copy(tot_v, out_hbm)


def _segment_sum_sc(e_p, ids_p):
    n_pad = e_p.shape[0]
    chunk = n_pad // NSC
    mesh = plsc.VectorSubcoreMesh(
        core_axis_name="c", subcore_axis_name="s", num_cores=1
    )
    run = functools.partial(
        pl.kernel,
        out_type=jax.ShapeDtypeStruct((N_SEG,), jnp.float32),
        mesh=mesh,
        compiler_params=pltpu.CompilerParams(needs_layout_passes=False),
        scratch_types=[
            pltpu.VMEM((chunk,), jnp.float32),
            pltpu.VMEM((chunk + LANES,), jnp.int32),
            pltpu.VMEM((ACC,), jnp.float32),
            pltpu.VMEM_SHARED((NSC * ACC,), jnp.float32),
            pltpu.VMEM((NSC * ACC,), jnp.float32),
            pltpu.VMEM((N_SEG,), jnp.float32),
        ],
    )(_sc_body)
    return run(e_p, ids_p)


def kernel(ind_1, elems, coord, W1, b1, W2, b2):
    n = coord.shape[0]
    ids = ind_1.reshape(-1).astype(jnp.int32)
    n_pad = -(-n // TILE) * TILE
    pad = n_pad - n
    coord_p = jnp.pad(coord, ((0, pad), (0, 0)))
    elems_p = jnp.pad(elems.astype(jnp.int32), (0, pad))
    ids_p = jnp.pad(ids, (0, pad), constant_values=N_SEG)
    e_p = _per_atom_energy(coord_p, elems_p, W1, b1, W2, b2)
    return _segment_sum_sc(e_p, ids_p)


# speedup vs baseline: 4.3899x; 2.2312x over previous
import functools

import jax
import jax.numpy as jnp
from jax import lax
from jax.experimental import pallas as pl
from jax.experimental.pallas import tpu as pltpu
from jax.experimental.pallas import tpu_sc as plsc

N_SEG = 512
TILE = 2048
NSC = 16
LANES = 16
ACC = N_SEG + LANES


def _tc_body(xt_ref, el_ref, w1t_ref, b1_ref, w2t_ref, b2_ref, e_ref):
    x = xt_ref[...]
    h = jnp.dot(w1t_ref[...], x, preferred_element_type=jnp.float32)
    t = jnp.tanh(h + b1_ref[...])
    ev = jnp.dot(w2t_ref[...], t, preferred_element_type=jnp.float32)
    el = el_ref[0]
    dress = (
        jnp.where(el == 1, jnp.float32(-0.5), jnp.float32(0.0))
        + jnp.where(el == 8, jnp.float32(-75.0), jnp.float32(0.0))
    )
    e_ref[0] = ev + b2_ref[...] + dress


def _per_atom_energy(coordT_p, elems3, W1t, b1c, W2t, b2c):
    n_pad = coordT_p.shape[1]
    hidden = W1t.shape[0]
    grid = n_pad // TILE
    out = pl.pallas_call(
        _tc_body,
        grid=(grid,),
        in_specs=[
            pl.BlockSpec((3, TILE), lambda i: (0, i)),
            pl.BlockSpec((1, 1, TILE), lambda i: (i, 0, 0)),
            pl.BlockSpec((hidden, 3), lambda i: (0, 0)),
            pl.BlockSpec((hidden, 1), lambda i: (0, 0)),
            pl.BlockSpec((1, hidden), lambda i: (0, 0)),
            pl.BlockSpec((1, 1), lambda i: (0, 0)),
        ],
        out_specs=pl.BlockSpec((1, 1, TILE), lambda i: (i, 0, 0)),
        out_shape=jax.ShapeDtypeStruct((grid, 1, TILE), jnp.float32),
    )(coordT_p, elems3, W1t, b1c, W2t, b2c)
    return out.reshape(n_pad)


def _sc_body(e_hbm, ids_hbm, out_hbm, e_v, ids_v, acc_v, shared, big_v, tot_v):
    chunk = e_v.shape[0]
    wid = lax.axis_index("s")
    base = wid * chunk
    pltpu.sync_copy(e_hbm.at[pl.ds(base, chunk)], e_v)
    pltpu.sync_copy(ids_hbm.at[pl.ds(base, chunk)], ids_v.at[pl.ds(0, chunk)])

    zero = jnp.zeros((LANES,), jnp.float32)
    for j in range(ACC // LANES):
        acc_v[pl.ds(j * LANES, LANES)] = zero

    iota = lax.iota(jnp.int32, LANES)
    last = LANES - 1

    def scatter_step(j, carry):
        off = j * LANES
        idc = ids_v[pl.ds(off, LANES)]
        idn = ids_v[pl.ds(off + 1, LANES)]
        val = e_v[pl.ds(off, LANES)]
        p = plsc.cumsum(val)
        bnd = (idc != idn) & (iota < last)
        end_mask = bnd | (iota == last)
        plsc.addupdate_scatter(acc_v, [idc], p, mask=end_mask)
        plsc.addupdate_scatter(acc_v, [idn], -p, mask=bnd)
        return carry

    lax.fori_loop(0, chunk // LANES, scatter_step, 0)

    pltpu.sync_copy(acc_v, shared.at[pl.ds(wid * ACC, ACC)])
    plsc.subcore_barrier()

    @pl.when(wid == 0)
    def _():
        pltpu.sync_copy(shared, big_v)
        for ch in range(N_SEG // LANES):
            s16 = big_v[pl.ds(ch * LANES, LANES)]
            for r in range(1, NSC):
                s16 = s16 + big_v[pl.ds(r * ACC + ch * LANES, LANES)]
            tot_v[pl.ds(ch * LANES, LANES)] = s16
        pltpu.sync_copy(tot_v, out_hbm)


def _segment_sum_sc(e_p, ids_p):
    n_pad = e_p.shape[0]
    chunk = n_pad // NSC
    mesh = plsc.VectorSubcoreMesh(
        core_axis_name="c", subcore_axis_name="s", num_cores=1
    )
    run = functools.partial(
        pl.kernel,
        out_type=jax.ShapeDtypeStruct((N_SEG,), jnp.float32),
        mesh=mesh,
        compiler_params=pltpu.CompilerParams(needs_layout_passes=False),
        scratch_types=[
            pltpu.VMEM((chunk,), jnp.float32),
            pltpu.VMEM((chunk + LANES,), jnp.int32),
            pltpu.VMEM((ACC,), jnp.float32),
            pltpu.VMEM_SHARED((NSC * ACC,), jnp.float32),
            pltpu.VMEM((NSC * ACC,), jnp.float32),
            pltpu.VMEM((N_SEG,), jnp.float32),
        ],
    )(_sc_body)
    return run(e_p, ids_p)


def kernel(ind_1, elems, coord, W1, b1, W2, b2):
    n = coord.shape[0]
    ids = ind_1.reshape(-1).astype(jnp.int32)
    n_pad = -(-n // TILE) * TILE
    pad = n_pad - n
    coordT_p = jnp.pad(coord.T, ((0, 0), (0, pad)))
    elems3 = jnp.pad(elems.astype(jnp.int32), (0, pad)).reshape(
        n_pad // TILE, 1, TILE
    )
    ids_p = jnp.pad(ids, (0, pad), constant_values=N_SEG)
    e_p = _per_atom_energy(
        coordT_p, elems3, W1.T, b1.reshape(-1, 1), W2.T, b2.reshape(1, 1)
    )
    return _segment_sum_sc(e_p, ids_p)
